# baseline (speedup 1.0000x reference)
import functools

import jax
import jax.numpy as jnp
from jax import lax
from jax.experimental import pallas as pl
from jax.experimental.pallas import tpu as pltpu
from jax.experimental.pallas import tpu_sc as plsc

NC = 2
NS = 16
NW = NC * NS

GROUP_B = 8


def _make_kernel(batch, n_fields, embed_dim):
    mesh = plsc.VectorSubcoreMesh(core_axis_name="c", subcore_axis_name="s")
    b_w = batch // NW
    n_groups = b_w // GROUP_B

    @functools.partial(
        pl.kernel,
        out_type=jax.ShapeDtypeStruct((batch, n_fields, embed_dim), jnp.float32),
        mesh=mesh,
        scratch_types=[
            pltpu.VMEM((b_w, n_fields), jnp.int32),
            pltpu.VMEM((2, GROUP_B * n_fields, embed_dim), jnp.float32),
            pltpu.SemaphoreType.DMA,
            pltpu.SemaphoreType.DMA,
            pltpu.SemaphoreType.DMA,
            pltpu.SemaphoreType.DMA,
        ],
        compiler_params=pltpu.CompilerParams(use_tc_tiling_on_sc=False),
    )
    def gather_kernel(table_hbm, idx_hbm, out_hbm, idx_v, rows_v,
                      sem_g0, sem_g1, sem_w0, sem_w1):
        wid = lax.axis_index("s") * NC + lax.axis_index("c")
        b_base = wid * b_w
        sem_g = (sem_g0, sem_g1)
        sem_w = (sem_w0, sem_w1)

        pltpu.sync_copy(idx_hbm.at[pl.ds(b_base, b_w)], idx_v)

        def gath(g, parity, j):
            return pltpu.make_async_copy(
                table_hbm.at[idx_v.at[g * GROUP_B + j]],
                rows_v.at[parity, pl.ds(j * n_fields, n_fields)],
                sem_g[parity],
            )

        def writ(g, parity, j):
            return pltpu.make_async_copy(
                rows_v.at[parity, pl.ds(j * n_fields, n_fields)],
                out_hbm.at[b_base + g * GROUP_B + j],
                sem_w[parity],
            )

        def fire(g, parity):
            for j in range(GROUP_B):
                gath(g, parity, j).start()

        def step(g, parity, fire_ahead):
            for j in range(GROUP_B):
                gath(g, parity, j).wait()
            for j in range(GROUP_B):
                writ(g, parity, j).start()
            if fire_ahead:
                for j in range(GROUP_B):
                    writ(g, parity, j).wait()
                fire(g + 2, parity)

        fire(0, 0)
        fire(1, 1)

        n_reg = n_groups - 3
        n_reg -= n_reg % 2

        def body(i, carry):
            g = i * 2
            step(g, 0, True)
            step(g + 1, 1, True)
            return carry

        lax.fori_loop(0, n_reg // 2, body, 0)

        for g in range(n_reg, n_groups):
            step(g, g % 2, g + 2 < n_groups)
        for g in (n_groups - 2, n_groups - 1):
            for j in range(GROUP_B):
                writ(g, g % 2, j).wait()

    return gather_kernel


def kernel(indices, table):
    batch, n_fields = indices.shape
    vocab, embed_dim = table.shape
    assert batch % (NW * GROUP_B) == 0
    return _make_kernel(batch, n_fields, embed_dim)(table, indices)


# speedup vs baseline: 1.0157x; 1.0157x over previous
import functools

import jax
import jax.numpy as jnp
from jax import lax
from jax.experimental import pallas as pl
from jax.experimental.pallas import tpu as pltpu
from jax.experimental.pallas import tpu_sc as plsc

NC = 2
NS = 16
NW = NC * NS

GROUP_B = 8


def _make_kernel(batch, n_fields, embed_dim):
    mesh = plsc.VectorSubcoreMesh(core_axis_name="c", subcore_axis_name="s")
    b_w = batch // NW
    n_groups = b_w // GROUP_B

    @functools.partial(
        pl.kernel,
        out_type=jax.ShapeDtypeStruct((batch, n_fields, embed_dim), jnp.float32),
        mesh=mesh,
        scratch_types=[
            pltpu.VMEM((b_w, n_fields), jnp.int32),
            pltpu.VMEM((2, GROUP_B * n_fields, embed_dim), jnp.float32),
            pltpu.SemaphoreType.DMA,
            pltpu.SemaphoreType.DMA,
            pltpu.SemaphoreType.DMA,
            pltpu.SemaphoreType.DMA,
        ],
        compiler_params=pltpu.CompilerParams(use_tc_tiling_on_sc=False),
    )
    def gather_kernel(table_hbm, idx_hbm, out_hbm, idx_v, rows_v,
                      sem_g0, sem_g1, sem_w0, sem_w1):
        wid = lax.axis_index("s") * NC + lax.axis_index("c")
        b_base = wid * b_w
        sem_g = (sem_g0, sem_g1)
        sem_w = (sem_w0, sem_w1)

        pltpu.sync_copy(idx_hbm.at[pl.ds(b_base, b_w)], idx_v)

        def gath(g, parity, j):
            return pltpu.make_async_copy(
                table_hbm.at[idx_v.at[g * GROUP_B + j]],
                rows_v.at[parity, pl.ds(j * n_fields, n_fields)],
                sem_g[parity],
            )

        def writ(g, parity, j):
            return pltpu.make_async_copy(
                rows_v.at[parity, pl.ds(j * n_fields, n_fields)],
                out_hbm.at[b_base + g * GROUP_B + j],
                sem_w[parity],
            )

        def fire(g, parity):
            for j in range(GROUP_B):
                gath(g, parity, j).start()

        def step(g, parity, fire_ahead):
            for j in range(GROUP_B):
                gath(g, parity, j).wait()
            for j in range(GROUP_B):
                writ(g, parity, j).start()
            if fire_ahead:
                for j in range(GROUP_B):
                    writ(g, parity, j).wait()
                fire(g + 2, parity)

        fire(0, 0)
        fire(1, 1)

        n_reg = n_groups - 3
        n_reg -= n_reg % 2

        def body(i, carry):
            g = i * 2
            step(g, 0, True)
            step(g + 1, 1, True)
            return carry

        lax.fori_loop(0, n_reg // 2, body, 0)

        for g in range(n_reg, n_groups):
            step(g, g % 2, g + 2 < n_groups)
        for g in (n_groups - 2, n_groups - 1):
            for j in range(GROUP_B):
                writ(g, g % 2, j).wait()

    return gather_kernel


def kernel(indices, table):
    batch, n_fields = indices.shape
    vocab, embed_dim = table.shape
    assert batch % (NW * GROUP_B) == 0
    pad = 128 // embed_dim
    tp = jnp.pad(table, ((0, 0), (0, (pad - 1) * embed_dim)))
    tp4 = tp.reshape(pad * vocab, embed_dim)
    return _make_kernel(batch, n_fields, embed_dim)(tp4, indices * pad)
